# Initial kernel scaffold; baseline (speedup 1.0000x reference)
import jax
import jax.numpy as jnp
from jax.experimental import pallas as pl


def kernel(x, edge_index, edge_attr, mlp_w, mlp_b, root_w, conv_bias, gn_weight, gn_bias, gn_mean_scale):
    raise NotImplementedError("write your pallas kernel here")


# speedup vs baseline: 3.0242x; 3.0242x over previous
import functools

import jax
import jax.numpy as jnp
from jax import lax
from jax.experimental import pallas as pl
from jax.experimental.pallas import tpu as pltpu
from jax.experimental.pallas import tpu_sc as plsc

_N = 10000
_E = 80000
_IN_C = 32
_OUT_C = 32
_ED = 16

_NC = 2
_NS = 16
_NW = _NC * _NS
_BATCH = 128
_NB = 20
_CHUNK = _NB * _BATCH
_E_PAD = _NW * _CHUNK
_PAD_SPREAD = 512
_N_SC = 10752
_ROWS_PER_TILE = _N_SC // _NS

_BE = 512
_N_BLOCKS = _E_PAD // _BE


@functools.cache
def _sc_kernels():
    mesh = plsc.VectorSubcoreMesh(core_axis_name="c", subcore_axis_name="s")

    @functools.partial(
        pl.kernel,
        out_type=jax.ShapeDtypeStruct((_NW, _NB, _BATCH, _IN_C), jnp.float32),
        mesh=mesh,
        scratch_types=[
            pltpu.VMEM((_NB, _BATCH), jnp.int32),
            pltpu.VMEM((_NB, _BATCH, _IN_C), jnp.float32),
            pltpu.SemaphoreType.DMA,
        ],
        compiler_params=pltpu.CompilerParams(use_tc_tiling_on_sc=False),
    )
    def sc_gather(x_hbm, src_hbm, out_hbm, idx_v, rows_v, sem):
        wid = lax.axis_index("s") * _NC + lax.axis_index("c")
        pltpu.sync_copy(src_hbm.at[wid], idx_v)
        descs = [
            pltpu.async_copy(x_hbm.at[idx_v.at[j]], rows_v.at[j], sem)
            for j in range(_NB)
        ]
        for d in descs:
            d.wait()
        pltpu.sync_copy(rows_v, out_hbm.at[wid])

    @functools.partial(
        pl.kernel,
        out_type=jax.ShapeDtypeStruct((_NC, _N_SC, _OUT_C), jnp.float32),
        mesh=mesh,
        scratch_types=[
            pltpu.VMEM((_NB, _BATCH), jnp.int32),
            pltpu.VMEM((_NB, _BATCH, _OUT_C), jnp.float32),
            pltpu.VMEM_SHARED((_N_SC, _OUT_C), jnp.float32),
            pltpu.SemaphoreType.DMA,
        ],
        compiler_params=pltpu.CompilerParams(use_tc_tiling_on_sc=False),
    )
    def sc_scatter(msgs_hbm, dst_hbm, zeros_hbm, out_hbm, idx_v, rows_v,
                   agg_sh, sem):
        c = lax.axis_index("c")
        s = lax.axis_index("s")
        wid = s * _NC + c
        row0 = s * _ROWS_PER_TILE
        pltpu.sync_copy(
            zeros_hbm.at[pl.ds(row0, _ROWS_PER_TILE)],
            agg_sh.at[pl.ds(row0, _ROWS_PER_TILE)],
        )
        pltpu.sync_copy(dst_hbm.at[wid], idx_v)
        pltpu.sync_copy(msgs_hbm.at[wid], rows_v)
        plsc.subcore_barrier()
        for j in range(_NB):
            pltpu.sync_copy(rows_v.at[j], agg_sh.at[idx_v.at[j]], add=True)
        plsc.subcore_barrier()
        pltpu.sync_copy(
            agg_sh.at[pl.ds(row0, _ROWS_PER_TILE)],
            out_hbm.at[c, pl.ds(row0, _ROWS_PER_TILE)],
        )

    return sc_gather, sc_scatter


def _msgs_body(eaT_ref, xj_ref, wT_ref, b_ref, out_ref):
    a = lax.dot_general(
        wT_ref[...], eaT_ref[...], (((1,), (0,)), ((), ())),
        preferred_element_type=jnp.float32,
    )
    a = jnp.maximum(a + b_ref[...], 0.0)
    xt = xj_ref[...].T
    acc = xt[0:1, :] * a[0:_OUT_C, :]
    for i in range(1, _IN_C):
        acc = acc + xt[i:i + 1, :] * a[i * _OUT_C:(i + 1) * _OUT_C, :]
    out_ref[...] = acc.T


def _final_body(p0_ref, p1_ref, x_ref, rw_ref, cb_ref, gw_ref, gb_ref,
                gms_ref, out_ref):
    x = x_ref[...]
    h = (p0_ref[...] + p1_ref[...]
         + jnp.dot(x, rw_ref[...], preferred_element_type=jnp.float32)
         + cb_ref[...])
    mean = jnp.mean(h, axis=0, keepdims=True)
    centered = h - gms_ref[...] * mean
    var = jnp.mean(centered * centered, axis=0, keepdims=True)
    hn = gw_ref[...] * centered * lax.rsqrt(var + 1e-5) + gb_ref[...]
    out_ref[...] = jnp.maximum(hn, 0.0) + x


def kernel(x, edge_index, edge_attr, mlp_w, mlp_b, root_w, conv_bias,
           gn_weight, gn_bias, gn_mean_scale):
    pad = _E_PAD - _E
    src = edge_index[0]
    dst = edge_index[1]
    pad_ids = jnp.arange(pad, dtype=jnp.int32)
    src_p = jnp.concatenate([src, pad_ids % _N]).reshape(_NW, _NB, _BATCH)
    dst_p = jnp.concatenate([dst, _N + pad_ids % _PAD_SPREAD]).reshape(
        _NW, _NB, _BATCH)

    sc_gather, sc_scatter = _sc_kernels()
    xj = sc_gather(x, src_p)
    xj = xj.reshape(_E_PAD, _IN_C)

    eaT = jnp.pad(edge_attr, ((0, pad), (0, 0))).T
    wT = mlp_w.T
    b_col = mlp_b.reshape(_IN_C * _OUT_C, 1)

    msgs = pl.pallas_call(
        _msgs_body,
        grid=(_N_BLOCKS,),
        in_specs=[
            pl.BlockSpec((_ED, _BE), lambda e: (0, e)),
            pl.BlockSpec((_BE, _IN_C), lambda e: (e, 0)),
            pl.BlockSpec((_IN_C * _OUT_C, _ED), lambda e: (0, 0)),
            pl.BlockSpec((_IN_C * _OUT_C, 1), lambda e: (0, 0)),
        ],
        out_specs=pl.BlockSpec((_BE, _OUT_C), lambda e: (e, 0)),
        out_shape=jax.ShapeDtypeStruct((_E_PAD, _OUT_C), jnp.float32),
    )(eaT, xj, wT, b_col)

    msgs3 = msgs.reshape(_NW, _NB, _BATCH, _OUT_C)
    zeros_sc = jnp.zeros((_N_SC, _OUT_C), jnp.float32)
    parts = sc_scatter(msgs3, dst_p, zeros_sc)

    out = pl.pallas_call(
        _final_body,
        in_specs=[pl.BlockSpec(s.shape, lambda: (0,) * len(s.shape))
                  for s in (
                      jax.ShapeDtypeStruct((_N, _OUT_C), jnp.float32),
                      jax.ShapeDtypeStruct((_N, _OUT_C), jnp.float32),
                      jax.ShapeDtypeStruct((_N, _IN_C), jnp.float32),
                      jax.ShapeDtypeStruct((_IN_C, _OUT_C), jnp.float32),
                      jax.ShapeDtypeStruct((1, _OUT_C), jnp.float32),
                      jax.ShapeDtypeStruct((1, _OUT_C), jnp.float32),
                      jax.ShapeDtypeStruct((1, _OUT_C), jnp.float32),
                      jax.ShapeDtypeStruct((1, _OUT_C), jnp.float32),
                  )],
        out_specs=pl.BlockSpec((_N, _OUT_C), lambda: (0, 0)),
        out_shape=jax.ShapeDtypeStruct((_N, _OUT_C), jnp.float32),
    )(
        parts[0, :_N],
        parts[1, :_N],
        x,
        root_w,
        conv_bias.reshape(1, _OUT_C),
        gn_weight.reshape(1, _OUT_C),
        gn_bias.reshape(1, _OUT_C),
        gn_mean_scale.reshape(1, _OUT_C),
    )
    return out
